# baseline (speedup 1.0000x reference)
import functools

import jax
import jax.numpy as jnp
import numpy as np
from jax import lax
from jax.experimental import pallas as pl
from jax.experimental.pallas import tpu as pltpu
from jax.experimental.pallas import tpu_sc as plsc

_NUM_OBJS = 1000
_NUM_RELS = 64
_BATCH = 16384

_NC, _NS, _L = 2, 16, 16
_NW = _NC * _NS
_B_PER_W = _BATCH // _NW
_CHUNK = 128
_N_CHUNKS = _B_PER_W // _CHUNK
_IPC = _CHUNK // _L


@functools.partial(
    pl.kernel,
    out_type=jax.ShapeDtypeStruct((_BATCH, _NUM_RELS), jnp.float32),
    mesh=plsc.VectorSubcoreMesh(core_axis_name="c", subcore_axis_name="s"),
    scratch_types=[
        pltpu.VMEM((2 * _B_PER_W,), jnp.int32),
        pltpu.VMEM((_N_CHUNKS, _CHUNK), jnp.int32),
        pltpu.VMEM((_B_PER_W, _NUM_RELS), jnp.float32),
        pltpu.SemaphoreType.DMA,
    ],
    compiler_params=pltpu.CompilerParams(use_tc_tiling_on_sc=False),
)
def _freq_bias_sc(labels_hbm, table_hbm, out_hbm, lab_v, idx_v, rows_v, sem):
    wid = lax.axis_index("s") * _NC + lax.axis_index("c")
    base = wid * _B_PER_W

    pltpu.sync_copy(labels_hbm.at[pl.ds(2 * base, 2 * _B_PER_W)], lab_v)

    lane = lax.iota(jnp.int32, _L)
    lo_half = lane < 8
    idx_ev = (lane & 7) * 2
    idx_od = idx_ev + 1

    def _vperm(v, idx):
        dnums = lax.GatherDimensionNumbers(
            offset_dims=(), collapsed_slice_dims=(0,), start_index_map=(0,))
        return lax.gather(v, idx[:, None], dnums, slice_sizes=(1,),
                          mode=lax.GatherScatterMode.PROMISE_IN_BOUNDS)

    for i in range(_B_PER_W // _L):
        u = lab_v[pl.ds(2 * i * _L, _L)]
        u2 = lab_v[pl.ds((2 * i + 1) * _L, _L)]
        l0 = jnp.where(lo_half, _vperm(u, idx_ev), _vperm(u2, idx_ev))
        l1 = jnp.where(lo_half, _vperm(u, idx_od), _vperm(u2, idx_od))
        idx_v[i // _IPC, pl.ds((i % _IPC) * _L, _L)] = l0 * _NUM_OBJS + l1

    copies = []
    for j in range(_N_CHUNKS):
        copies.append(
            pltpu.async_copy(
                table_hbm.at[idx_v.at[j]],
                rows_v.at[pl.ds(j * _CHUNK, _CHUNK), :],
                sem,
            )
        )
    for c in copies:
        c.wait()

    pltpu.sync_copy(rows_v, out_hbm.at[pl.ds(base, _B_PER_W), :])


def kernel(labels, obj_baseline):
    return _freq_bias_sc(labels.reshape(-1), obj_baseline)


# speedup vs baseline: 2.4322x; 2.4322x over previous
import functools

import jax
import jax.numpy as jnp
from jax import lax
from jax.experimental import pallas as pl
from jax.experimental.pallas import tpu as pltpu
from jax.experimental.pallas import tpu_sc as plsc

_NUM_OBJS = 1000
_NUM_RELS = 64
_BATCH = 16384
_TBL_COLS = _NUM_OBJS * _NUM_OBJS

_NC, _NS, _L = 2, 16, 16
_NW = _NC * _NS
_B_PER_W = _BATCH // _NW
_PAN = 128
_K = 4
_NGROUPS = _B_PER_W // _K
_GPB = _PAN // _K


@functools.partial(
    pl.kernel,
    out_type=jax.ShapeDtypeStruct((_NUM_RELS, _BATCH), jnp.float32),
    mesh=plsc.VectorSubcoreMesh(core_axis_name="c", subcore_axis_name="s"),
    scratch_types=[
        pltpu.VMEM((2 * _B_PER_W,), jnp.int32),
        pltpu.VMEM((_B_PER_W,), jnp.int32),
        pltpu.VMEM((2, _K, _NUM_RELS, _PAN), jnp.float32),
        pltpu.VMEM((_NUM_RELS, _PAN), jnp.float32),
        pltpu.SemaphoreType.DMA((2,)),
    ],
    compiler_params=pltpu.CompilerParams(
        use_tc_tiling_on_sc=True,
        disable_bounds_checks=True,
        needs_layout_passes=False,
    ),
)
def _freq_bias_sc(labels_hbm, tbl_t_hbm, out_t_hbm,
                  lab_v, idx_v, slab_v, blk_v, sem):
    wid = lax.axis_index("s") * _NC + lax.axis_index("c")
    base = wid * _B_PER_W

    pltpu.sync_copy(labels_hbm.at[pl.ds(2 * base, 2 * _B_PER_W)], lab_v)

    lane = lax.iota(jnp.int32, _L)
    lo_half = lane < 8
    idx_ev = (lane & 7) * 2
    idx_od = idx_ev + 1

    def _vperm(v, idx):
        dnums = lax.GatherDimensionNumbers(
            offset_dims=(), collapsed_slice_dims=(0,), start_index_map=(0,))
        return lax.gather(v, idx[:, None], dnums, slice_sizes=(1,),
                          mode=lax.GatherScatterMode.PROMISE_IN_BOUNDS)

    for i in range(_B_PER_W // _L):
        u = lab_v[pl.ds(2 * i * _L, _L)]
        u2 = lab_v[pl.ds((2 * i + 1) * _L, _L)]
        l0 = jnp.where(lo_half, _vperm(u, idx_ev), _vperm(u2, idx_ev))
        l1 = jnp.where(lo_half, _vperm(u, idx_od), _vperm(u2, idx_od))
        idx_v[pl.ds(i * _L, _L)] = l0 * _NUM_OBJS + l1

    def flat_idx(j):
        chunk = idx_v[pl.ds((j // _L) * _L, _L)]
        return jnp.sum(jnp.where(lane == lax.rem(j, _L), chunk, 0))

    def fire(g):
        slot = lax.rem(g, 2)
        for q in range(_K):
            r = flat_idx(g * _K + q)
            cbase = pl.multiple_of((r >> 7) * _PAN, _PAN)
            pltpu.async_copy(
                tbl_t_hbm.at[:, pl.ds(cbase, _PAN)],
                slab_v.at[slot, q], sem.at[slot])

    fire(jnp.int32(0))

    def body(g, _):
        slot = lax.rem(g, 2)

        @pl.when(g + 1 < _NGROUPS)
        def _():
            fire(g + 1)

        for q in range(_K):
            pltpu.make_async_copy(
                tbl_t_hbm.at[:, pl.ds(0, _PAN)],
                slab_v.at[slot, q], sem.at[slot]).wait()

        for q in range(_K):
            j = g * _K + q
            r = flat_idx(j)
            rr = r & (_PAN - 1)
            jj = j & (_PAN - 1)
            rrv = jnp.broadcast_to(rr, (_L,))
            jjv = jnp.broadcast_to(jj, (_L,))
            for k in range(_NUM_RELS // _L):
                dvec = lane + k * _L
                col = plsc.load_gather(slab_v.at[slot, q], [dvec, rrv])
                plsc.store_scatter(blk_v, [dvec, jjv], col)

        @pl.when(lax.rem(g, _GPB) == _GPB - 1)
        def _():
            boff = pl.multiple_of(base + (g // _GPB) * _PAN, _PAN)
            pltpu.sync_copy(blk_v, out_t_hbm.at[:, pl.ds(boff, _PAN)])

        return 0

    lax.fori_loop(0, _NGROUPS, body, 0)


def kernel(labels, obj_baseline):
    out_t = _freq_bias_sc(labels.reshape(-1), obj_baseline.T)
    return out_t.T


# speedup vs baseline: 2.6630x; 1.0949x over previous
import functools

import jax
import jax.numpy as jnp
from jax import lax
from jax.experimental import pallas as pl
from jax.experimental.pallas import tpu as pltpu
from jax.experimental.pallas import tpu_sc as plsc

_NUM_OBJS = 1000
_NUM_RELS = 64
_BATCH = 16384
_TBL_COLS = _NUM_OBJS * _NUM_OBJS

_NC, _NS, _L = 2, 16, 16
_NW = _NC * _NS
_B_PER_W = _BATCH // _NW
_PAN = 128
_K = 4
_NGROUPS = _B_PER_W // _K
_GPB = _PAN // _K


@functools.partial(
    pl.kernel,
    out_type=jax.ShapeDtypeStruct((_NUM_RELS, _BATCH), jnp.float32),
    mesh=plsc.VectorSubcoreMesh(core_axis_name="c", subcore_axis_name="s"),
    scratch_types=[
        pltpu.VMEM((2 * _B_PER_W,), jnp.int32),
        pltpu.VMEM((_B_PER_W,), jnp.int32),
        pltpu.VMEM((3, _K, _NUM_RELS, _PAN), jnp.float32),
        pltpu.VMEM((_NUM_RELS, _PAN), jnp.float32),
        pltpu.SemaphoreType.DMA((3,)),
    ],
    compiler_params=pltpu.CompilerParams(
        use_tc_tiling_on_sc=True,
        disable_bounds_checks=True,
        needs_layout_passes=False,
    ),
)
def _freq_bias_sc(labels_hbm, tbl_t_hbm, out_t_hbm,
                  lab_v, idx_v, slab_v, blk_v, sem):
    wid = lax.axis_index("s") * _NC + lax.axis_index("c")
    base = wid * _B_PER_W

    pltpu.sync_copy(labels_hbm.at[pl.ds(2 * base, 2 * _B_PER_W)], lab_v)

    lane = lax.iota(jnp.int32, _L)
    lo_half = lane < 8
    idx_ev = (lane & 7) * 2
    idx_od = idx_ev + 1

    def _vperm(v, idx):
        dnums = lax.GatherDimensionNumbers(
            offset_dims=(), collapsed_slice_dims=(0,), start_index_map=(0,))
        return lax.gather(v, idx[:, None], dnums, slice_sizes=(1,),
                          mode=lax.GatherScatterMode.PROMISE_IN_BOUNDS)

    for i in range(_B_PER_W // _L):
        u = lab_v[pl.ds(2 * i * _L, _L)]
        u2 = lab_v[pl.ds((2 * i + 1) * _L, _L)]
        l0 = jnp.where(lo_half, _vperm(u, idx_ev), _vperm(u2, idx_ev))
        l1 = jnp.where(lo_half, _vperm(u, idx_od), _vperm(u2, idx_od))
        idx_v[pl.ds(i * _L, _L)] = l0 * _NUM_OBJS + l1

    def flat_idx(j):
        chunk = idx_v[pl.ds((j // _L) * _L, _L)]
        return jnp.sum(jnp.where(lane == lax.rem(j, _L), chunk, 0))

    def fire(g):
        slot = lax.rem(g, 3)
        for q in range(_K):
            r = flat_idx(g * _K + q)
            cbase = pl.multiple_of((r >> 7) * _PAN, _PAN)
            pltpu.async_copy(
                tbl_t_hbm.at[:, pl.ds(cbase, _PAN)],
                slab_v.at[slot, q], sem.at[slot])

    fire(jnp.int32(0))
    fire(jnp.int32(1))

    def body(g, _):
        slot = lax.rem(g, 3)

        @pl.when(g + 2 < _NGROUPS)
        def _():
            fire(g + 2)

        for q in range(_K):
            pltpu.make_async_copy(
                tbl_t_hbm.at[:, pl.ds(0, _PAN)],
                slab_v.at[slot, q], sem.at[slot]).wait()

        for q in range(_K):
            j = g * _K + q
            r = flat_idx(j)
            rr = r & (_PAN - 1)
            jj = j & (_PAN - 1)
            rrv = jnp.broadcast_to(rr, (_L,))
            jjv = jnp.broadcast_to(jj, (_L,))
            for k in range(_NUM_RELS // _L):
                dvec = lane + k * _L
                col = plsc.load_gather(slab_v.at[slot, q], [dvec, rrv])
                plsc.store_scatter(blk_v, [dvec, jjv], col)

        @pl.when(lax.rem(g, _GPB) == _GPB - 1)
        def _():
            boff = pl.multiple_of(base + (g // _GPB) * _PAN, _PAN)
            pltpu.sync_copy(blk_v, out_t_hbm.at[:, pl.ds(boff, _PAN)])

        return 0

    lax.fori_loop(0, _NGROUPS, body, 0)


def kernel(labels, obj_baseline):
    out_t = _freq_bias_sc(labels.reshape(-1), obj_baseline.T)
    return out_t.T


# speedup vs baseline: 2.9130x; 1.0939x over previous
import functools

import jax
import jax.numpy as jnp
from jax import lax
from jax.experimental import pallas as pl
from jax.experimental.pallas import tpu as pltpu
from jax.experimental.pallas import tpu_sc as plsc

_NUM_OBJS = 1000
_NUM_RELS = 64
_BATCH = 16384
_TBL_COLS = _NUM_OBJS * _NUM_OBJS

_NC, _NS, _L = 2, 16, 16
_NW = _NC * _NS
_B_PER_W = _BATCH // _NW
_PAN = 128
_RING = 14
_LEAD = _RING - 1


@functools.partial(
    pl.kernel,
    out_type=jax.ShapeDtypeStruct((_NUM_RELS, _BATCH), jnp.float32),
    mesh=plsc.VectorSubcoreMesh(core_axis_name="c", subcore_axis_name="s"),
    scratch_types=[
        pltpu.VMEM((2 * _B_PER_W,), jnp.int32),
        pltpu.VMEM((_B_PER_W,), jnp.int32),
        pltpu.VMEM((_RING, _NUM_RELS, _PAN), jnp.float32),
        pltpu.VMEM((_NUM_RELS, _PAN), jnp.float32),
        pltpu.SemaphoreType.DMA((_RING,)),
    ],
    compiler_params=pltpu.CompilerParams(
        use_tc_tiling_on_sc=True,
        disable_bounds_checks=True,
        needs_layout_passes=False,
    ),
)
def _freq_bias_sc(labels_hbm, tbl_t_hbm, out_t_hbm,
                  lab_v, idx_v, slab_v, blk_v, sem):
    wid = lax.axis_index("s") * _NC + lax.axis_index("c")
    base = wid * _B_PER_W

    pltpu.sync_copy(labels_hbm.at[pl.ds(2 * base, 2 * _B_PER_W)], lab_v)

    lane = lax.iota(jnp.int32, _L)
    lo_half = lane < 8
    idx_ev = (lane & 7) * 2
    idx_od = idx_ev + 1

    def _vperm(v, idx):
        dnums = lax.GatherDimensionNumbers(
            offset_dims=(), collapsed_slice_dims=(0,), start_index_map=(0,))
        return lax.gather(v, idx[:, None], dnums, slice_sizes=(1,),
                          mode=lax.GatherScatterMode.PROMISE_IN_BOUNDS)

    for i in range(_B_PER_W // _L):
        u = lab_v[pl.ds(2 * i * _L, _L)]
        u2 = lab_v[pl.ds((2 * i + 1) * _L, _L)]
        l0 = jnp.where(lo_half, _vperm(u, idx_ev), _vperm(u2, idx_ev))
        l1 = jnp.where(lo_half, _vperm(u, idx_od), _vperm(u2, idx_od))
        idx_v[pl.ds(i * _L, _L)] = l0 * _NUM_OBJS + l1

    def flat_idx(j):
        chunk = idx_v[pl.ds((j // _L) * _L, _L)]
        return jnp.sum(jnp.where(lane == lax.rem(j, _L), chunk, 0))

    def fire(j):
        slot = lax.rem(j, _RING)
        r = flat_idx(j)
        cbase = pl.multiple_of((r >> 7) * _PAN, _PAN)
        pltpu.async_copy(
            tbl_t_hbm.at[:, pl.ds(cbase, _PAN)],
            slab_v.at[slot], sem.at[slot])

    for j0 in range(_LEAD):
        fire(jnp.int32(j0))

    def body(j, _):
        slot = lax.rem(j, _RING)

        @pl.when(j + _LEAD < _B_PER_W)
        def _():
            fire(j + _LEAD)

        pltpu.make_async_copy(
            tbl_t_hbm.at[:, pl.ds(0, _PAN)],
            slab_v.at[slot], sem.at[slot]).wait()

        r = flat_idx(j)
        rr = r & (_PAN - 1)
        jj = j & (_PAN - 1)
        rrv = jnp.broadcast_to(rr, (_L,))
        jjv = jnp.broadcast_to(jj, (_L,))
        for k in range(_NUM_RELS // _L):
            dvec = lane + k * _L
            col = plsc.load_gather(slab_v.at[slot], [dvec, rrv])
            plsc.store_scatter(blk_v, [dvec, jjv], col)

        @pl.when(lax.rem(j, _PAN) == _PAN - 1)
        def _():
            boff = pl.multiple_of(base + (j // _PAN) * _PAN, _PAN)
            pltpu.sync_copy(blk_v, out_t_hbm.at[:, pl.ds(boff, _PAN)])

        return 0

    lax.fori_loop(0, _B_PER_W, body, 0)


def kernel(labels, obj_baseline):
    out_t = _freq_bias_sc(labels.reshape(-1), obj_baseline.T)
    return out_t.T


# speedup vs baseline: 3.6656x; 1.2584x over previous
import functools

import jax
import jax.numpy as jnp
from jax import lax
from jax.experimental import pallas as pl
from jax.experimental.pallas import tpu as pltpu
from jax.experimental.pallas import tpu_sc as plsc

_NUM_OBJS = 1000
_NUM_RELS = 64
_BATCH = 16384
_TBL_COLS = _NUM_OBJS * _NUM_OBJS

_NC, _NS, _L = 2, 16, 16
_NW = _NC * _NS
_B_PER_W = _BATCH // _NW
_PAN = 128
_RING = 10
_LEAD = _RING - 1
_ND_CAP = _B_PER_W + _L


@functools.partial(
    pl.kernel,
    out_type=jax.ShapeDtypeStruct((_NUM_RELS, _BATCH), jnp.float32),
    mesh=plsc.VectorSubcoreMesh(core_axis_name="c", subcore_axis_name="s"),
    scratch_types=[
        pltpu.VMEM((2 * _BATCH,), jnp.int32),
        pltpu.VMEM((_B_PER_W,), jnp.int32),
        pltpu.VMEM((_B_PER_W,), jnp.int32),
        pltpu.VMEM((_ND_CAP,), jnp.int32),
        pltpu.VMEM((_ND_CAP,), jnp.int32),
        pltpu.VMEM((_RING, _NUM_RELS, _PAN), jnp.float32),
        pltpu.VMEM((_NUM_RELS, _PAN), jnp.float32),
        pltpu.SemaphoreType.DMA((_RING,)),
    ],
    compiler_params=pltpu.CompilerParams(
        use_tc_tiling_on_sc=True,
        disable_bounds_checks=True,
        needs_layout_passes=False,
    ),
)
def _gather_sorted_sc(labels_hbm, tbl_t_hbm, perm_hbm, out_t_hbm,
                      lab_v, perm_v, flat_v, dpan_v, dstart_v,
                      slab_v, blk_v, sem):
    wid = lax.axis_index("s") * _NC + lax.axis_index("c")
    base = wid * _B_PER_W

    pltpu.sync_copy(labels_hbm, lab_v)
    pltpu.sync_copy(perm_hbm.at[pl.ds(base, _B_PER_W)], perm_v)

    lane = lax.iota(jnp.int32, _L)
    lanem1 = jnp.maximum(lane - 1, 0)

    def _vperm(v, idx):
        dnums = lax.GatherDimensionNumbers(
            offset_dims=(), collapsed_slice_dims=(0,), start_index_map=(0,))
        return lax.gather(v, idx[:, None], dnums, slice_sizes=(1,),
                          mode=lax.GatherScatterMode.PROMISE_IN_BOUNDS)

    def _lane15(v):
        return jnp.sum(jnp.where(lane == _L - 1, v, 0))

    def _scalar_read(ref, j):
        chunk = ref[pl.ds((j // _L) * _L, _L)]
        return jnp.sum(jnp.where(lane == lax.rem(j, _L), chunk, 0))

    n_d = jnp.int32(0)
    carry = jnp.int32(-1)
    for i in range(_B_PER_W // _L):
        p = perm_v[pl.ds(i * _L, _L)]
        l0 = plsc.load_gather(lab_v, [p * 2])
        l1 = plsc.load_gather(lab_v, [p * 2 + 1])
        fl = l0 * _NUM_OBJS + l1
        flat_v[pl.ds(i * _L, _L)] = fl
        pan = fl >> 7
        shifted = jnp.where(lane == 0, carry, _vperm(pan, lanem1))
        m = pan != shifted
        d = n_d + jnp.cumsum(m.astype(jnp.int32)) - 1
        plsc.store_scatter(dpan_v, [d], pan, mask=m)
        plsc.store_scatter(dstart_v, [d], lane + i * _L, mask=m)
        n_d = _lane15(d) + 1
        carry = _lane15(pan)

    plsc.store_scatter(dstart_v, [jnp.broadcast_to(n_d, (_L,))],
                       jnp.broadcast_to(jnp.int32(_B_PER_W), (_L,)),
                       mask=lane == 0)

    def fire(d):
        slot = lax.rem(d, _RING)
        panel = _scalar_read(dpan_v, d)
        cbase = pl.multiple_of(panel * _PAN, _PAN)
        pltpu.async_copy(
            tbl_t_hbm.at[:, pl.ds(cbase, _PAN)],
            slab_v.at[slot], sem.at[slot])

    for d0 in range(_LEAD):
        @pl.when(d0 < n_d)
        def _():
            fire(jnp.int32(d0))

    def body(d, _):
        slot = lax.rem(d, _RING)

        @pl.when(d + _LEAD < n_d)
        def _():
            fire(d + _LEAD)

        pltpu.make_async_copy(
            tbl_t_hbm.at[:, pl.ds(0, _PAN)],
            slab_v.at[slot], sem.at[slot]).wait()

        j0 = _scalar_read(dstart_v, d)
        j1 = _scalar_read(dstart_v, d + 1)

        def item_body(j, _):
            r = _scalar_read(flat_v, j)
            rr = r & (_PAN - 1)
            jj = j & (_PAN - 1)
            rrv = jnp.broadcast_to(rr, (_L,))
            jjv = jnp.broadcast_to(jj, (_L,))
            for k in range(_NUM_RELS // _L):
                dvec = lane + k * _L
                col = plsc.load_gather(slab_v.at[slot], [dvec, rrv])
                plsc.store_scatter(blk_v, [dvec, jjv], col)

            @pl.when(jj == _PAN - 1)
            def _():
                boff = pl.multiple_of(base + (j // _PAN) * _PAN, _PAN)
                pltpu.sync_copy(blk_v, out_t_hbm.at[:, pl.ds(boff, _PAN)])

            return 0

        lax.fori_loop(j0, j1, item_body, 0)
        return 0

    lax.fori_loop(0, n_d, body, 0)


_INV_CHUNKS = _B_PER_W // _PAN


@functools.partial(
    pl.kernel,
    out_type=jax.ShapeDtypeStruct((_BATCH, _NUM_RELS), jnp.float32),
    mesh=plsc.VectorSubcoreMesh(core_axis_name="c", subcore_axis_name="s"),
    scratch_types=[
        pltpu.VMEM((_INV_CHUNKS, _PAN), jnp.int32),
        pltpu.VMEM((_B_PER_W, _NUM_RELS), jnp.float32),
        pltpu.SemaphoreType.DMA,
    ],
    compiler_params=pltpu.CompilerParams(use_tc_tiling_on_sc=False),
)
def _unpermute_sc(rows_hbm, inv_hbm, out_hbm, inv_v, rows_v, sem):
    wid = lax.axis_index("s") * _NC + lax.axis_index("c")
    base = wid * _B_PER_W

    for j in range(_INV_CHUNKS):
        pltpu.sync_copy(inv_hbm.at[pl.ds(base + j * _PAN, _PAN)], inv_v.at[j])

    copies = []
    for j in range(_INV_CHUNKS):
        copies.append(
            pltpu.async_copy(
                rows_hbm.at[inv_v.at[j]],
                rows_v.at[pl.ds(j * _PAN, _PAN), :],
                sem))
    for c in copies:
        c.wait()

    pltpu.sync_copy(rows_v, out_hbm.at[pl.ds(base, _B_PER_W), :])


def kernel(labels, obj_baseline):
    labels = labels.astype(jnp.int32)
    flat = labels[:, 0] * _NUM_OBJS + labels[:, 1]
    perm = jnp.argsort(flat).astype(jnp.int32)
    inv = jnp.zeros((_BATCH,), jnp.int32).at[perm].set(
        jnp.arange(_BATCH, dtype=jnp.int32))
    out2_t = _gather_sorted_sc(labels.reshape(-1), obj_baseline.T, perm)
    return _unpermute_sc(out2_t.T, inv)


# speedup vs baseline: 3.6670x; 1.0004x over previous
import functools

import jax
import jax.numpy as jnp
from jax import lax
from jax.experimental import pallas as pl
from jax.experimental.pallas import tpu as pltpu
from jax.experimental.pallas import tpu_sc as plsc

_NUM_OBJS = 1000
_NUM_RELS = 64
_BATCH = 16384
_TBL_COLS = _NUM_OBJS * _NUM_OBJS

_NC, _NS, _L = 2, 16, 16
_NW = _NC * _NS
_B_PER_W = _BATCH // _NW
_PAN = 128
_RING = 10
_LEAD = _RING - 1
_ND_CAP = _B_PER_W + _L


@functools.partial(
    pl.kernel,
    out_type=jax.ShapeDtypeStruct((_NUM_RELS, _BATCH), jnp.float32),
    mesh=plsc.VectorSubcoreMesh(core_axis_name="c", subcore_axis_name="s"),
    scratch_types=[
        pltpu.VMEM((2 * _BATCH,), jnp.int32),
        pltpu.VMEM((_B_PER_W,), jnp.int32),
        pltpu.VMEM((_B_PER_W,), jnp.int32),
        pltpu.VMEM((_ND_CAP,), jnp.int32),
        pltpu.VMEM((_ND_CAP,), jnp.int32),
        pltpu.VMEM((_RING, _NUM_RELS, _PAN), jnp.float32),
        pltpu.VMEM((_NUM_RELS, _PAN), jnp.float32),
        pltpu.SemaphoreType.DMA((_RING,)),
    ],
    compiler_params=pltpu.CompilerParams(
        use_tc_tiling_on_sc=True,
        disable_bounds_checks=True,
        needs_layout_passes=False,
    ),
)
def _gather_sorted_sc(labels_hbm, tbl_t_hbm, perm_hbm, out_t_hbm,
                      lab_v, perm_v, flat_v, dpan_v, dstart_v,
                      slab_v, blk_v, sem):
    wid = lax.axis_index("s") * _NC + lax.axis_index("c")
    base = wid * _B_PER_W

    pltpu.sync_copy(labels_hbm, lab_v)
    pltpu.sync_copy(perm_hbm.at[pl.ds(base, _B_PER_W)], perm_v)

    lane = lax.iota(jnp.int32, _L)
    lanem1 = jnp.maximum(lane - 1, 0)

    def _vperm(v, idx):
        dnums = lax.GatherDimensionNumbers(
            offset_dims=(), collapsed_slice_dims=(0,), start_index_map=(0,))
        return lax.gather(v, idx[:, None], dnums, slice_sizes=(1,),
                          mode=lax.GatherScatterMode.PROMISE_IN_BOUNDS)

    def _lane15(v):
        return jnp.sum(jnp.where(lane == _L - 1, v, 0))

    def _scalar_read(ref, j):
        chunk = ref[pl.ds((j // _L) * _L, _L)]
        return jnp.sum(jnp.where(lane == lax.rem(j, _L), chunk, 0))

    n_d = jnp.int32(0)
    carry = jnp.int32(-1)
    for i in range(_B_PER_W // _L):
        p = perm_v[pl.ds(i * _L, _L)]
        l0 = plsc.load_gather(lab_v, [p * 2])
        l1 = plsc.load_gather(lab_v, [p * 2 + 1])
        fl = l0 * _NUM_OBJS + l1
        flat_v[pl.ds(i * _L, _L)] = fl
        pan = fl >> 7
        shifted = jnp.where(lane == 0, carry, _vperm(pan, lanem1))
        m = pan != shifted
        d = n_d + jnp.cumsum(m.astype(jnp.int32)) - 1
        plsc.store_scatter(dpan_v, [d], pan, mask=m)
        plsc.store_scatter(dstart_v, [d], lane + i * _L, mask=m)
        n_d = _lane15(d) + 1
        carry = _lane15(pan)

    plsc.store_scatter(dstart_v, [jnp.broadcast_to(n_d, (_L,))],
                       jnp.broadcast_to(jnp.int32(_B_PER_W), (_L,)),
                       mask=lane == 0)

    def fire(d):
        slot = lax.rem(d, _RING)
        panel = _scalar_read(dpan_v, d)
        cbase = pl.multiple_of(panel * _PAN, _PAN)
        pltpu.async_copy(
            tbl_t_hbm.at[:, pl.ds(cbase, _PAN)],
            slab_v.at[slot], sem.at[slot])

    for d0 in range(_LEAD):
        @pl.when(d0 < n_d)
        def _():
            fire(jnp.int32(d0))

    def body(d, _):
        slot = lax.rem(d, _RING)

        @pl.when(d + _LEAD < n_d)
        def _():
            fire(d + _LEAD)

        pltpu.make_async_copy(
            tbl_t_hbm.at[:, pl.ds(0, _PAN)],
            slab_v.at[slot], sem.at[slot]).wait()

        j0 = _scalar_read(dstart_v, d)
        j1 = _scalar_read(dstart_v, d + 1)

        def item_body(j, _):
            r = _scalar_read(flat_v, j)
            rr = r & (_PAN - 1)
            jj = j & (_PAN - 1)
            rrv = jnp.broadcast_to(rr, (_L,))
            jjv = jnp.broadcast_to(jj, (_L,))
            for k in range(_NUM_RELS // _L):
                dvec = lane + k * _L
                col = plsc.load_gather(slab_v.at[slot], [dvec, rrv])
                plsc.store_scatter(blk_v, [dvec, jjv], col)

            @pl.when(jj == _PAN - 1)
            def _():
                boff = pl.multiple_of(base + (j // _PAN) * _PAN, _PAN)
                pltpu.sync_copy(blk_v, out_t_hbm.at[:, pl.ds(boff, _PAN)])

            return 0

        lax.fori_loop(j0, j1, item_body, 0)
        return 0

    lax.fori_loop(0, n_d, body, 0)


_INV_CHUNKS = _B_PER_W // _PAN


@functools.partial(
    pl.kernel,
    out_type=jax.ShapeDtypeStruct((_BATCH, _NUM_RELS), jnp.float32),
    mesh=plsc.VectorSubcoreMesh(core_axis_name="c", subcore_axis_name="s"),
    scratch_types=[
        pltpu.VMEM((_INV_CHUNKS, _PAN), jnp.int32),
        pltpu.VMEM((_B_PER_W, _NUM_RELS), jnp.float32),
        pltpu.SemaphoreType.DMA,
    ],
    compiler_params=pltpu.CompilerParams(use_tc_tiling_on_sc=False),
)
def _unpermute_sc(rows_hbm, perm_hbm, out_hbm, perm_v, rows_v, sem):
    wid = lax.axis_index("s") * _NC + lax.axis_index("c")
    base = wid * _B_PER_W

    for j in range(_INV_CHUNKS):
        pltpu.sync_copy(perm_hbm.at[pl.ds(base + j * _PAN, _PAN)], perm_v.at[j])

    pltpu.sync_copy(rows_hbm.at[pl.ds(base, _B_PER_W), :], rows_v)

    copies = []
    for j in range(_INV_CHUNKS):
        copies.append(
            pltpu.async_copy(
                rows_v.at[pl.ds(j * _PAN, _PAN), :],
                out_hbm.at[perm_v.at[j]],
                sem))
    for c in copies:
        c.wait()


def kernel(labels, obj_baseline):
    labels = labels.astype(jnp.int32)
    flat = labels[:, 0] * _NUM_OBJS + labels[:, 1]
    key = (flat >> 7) * _BATCH + jnp.arange(_BATCH, dtype=jnp.int32)
    perm = jnp.sort(key) & (_BATCH - 1)
    out2_t = _gather_sorted_sc(labels.reshape(-1), obj_baseline.T, perm)
    return _unpermute_sc(out2_t.T, perm)
